# Initial kernel scaffold; baseline (speedup 1.0000x reference)
import jax
import jax.numpy as jnp
from jax.experimental import pallas as pl


def kernel(features, labels, W, b, proto, temperature):
    raise NotImplementedError("write your pallas kernel here")


# speedup vs baseline: 1.4999x; 1.4999x over previous
import functools

import jax
import jax.numpy as jnp
from jax import lax
from jax.experimental import pallas as pl
from jax.experimental.pallas import tpu as pltpu

B_, S_, D_, T_ = 4, 2048, 1024, 128
N_ = B_ * S_
TOK_TILE = 1024
N_TILES = N_ // TOK_TILE
LAB_ROWS = N_ // 1024
EPS = 1e-8
INV_LN2 = 1.4426950408889634


def _segsum_body(lab_ref, x_ref, s_ref):
    pid = pl.program_id(0)

    @pl.when(pid == 0)
    def _():
        s_ref[...] = jnp.zeros_like(s_ref)

    lab_row = lab_ref[pl.ds(pid, 1), :]
    tag_iota = lax.broadcasted_iota(jnp.int32, (T_, 1), 0)
    onehot_t = (lab_row == tag_iota).astype(jnp.float32)
    s_ref[...] += lax.dot_general(
        onehot_t, x_ref[...], (((1,), (0,)), ((), ())),
        preferred_element_type=jnp.float32)


def _epilogue_body(s_ref, lab_ref, w_ref, b_ref, proto_ref, temp_ref, out_ref):
    S = s_ref[...]
    temp = temp_ref[0, 0]

    tag_iota = lax.broadcasted_iota(jnp.int32, (T_, 1), 0)
    counts = jnp.zeros((T_, 1), jnp.float32)
    for c in range(LAB_ROWS):
        row = lab_ref[c:c + 1, :]
        oh = (row == tag_iota).astype(jnp.float32)
        counts = counts + jnp.sum(oh, axis=1, keepdims=True)

    sums = lax.dot_general(
        S, w_ref[...], (((1,), (1,)), ((), ())),
        precision=lax.Precision.HIGHEST,
        preferred_element_type=jnp.float32)
    sums = sums + counts * b_ref[...]
    means = sums / jnp.maximum(counts, 1.0)

    mn = means / jnp.maximum(
        jnp.sqrt(jnp.sum(means * means, axis=1, keepdims=True)), EPS)
    proto = proto_ref[...]
    pn = proto / jnp.maximum(
        jnp.sqrt(jnp.sum(proto * proto, axis=1, keepdims=True)), EPS)

    cos_t = lax.dot_general(
        pn, mn, (((1,), (1,)), ((), ())),
        precision=lax.Precision.HIGHEST,
        preferred_element_type=jnp.float32)
    ap_t = -(1.0 - cos_t) / temp

    sim = lax.dot_general(
        pn, pn, (((1,), (1,)), ((), ())),
        precision=lax.Precision.HIGHEST,
        preferred_element_type=jnp.float32)

    BK = 8
    rank_t = jnp.zeros((T_, T_), jnp.float32)
    sim3 = sim[None, :, :]
    jmat = lax.broadcasted_iota(jnp.int32, (BK, T_, T_), 1)
    for kb in range(T_ // BK):
        blk = sim[kb * BK:(kb + 1) * BK, :]
        blk3 = blk[:, None, :]
        kvec = kb * BK + lax.broadcasted_iota(jnp.int32, (BK, T_, T_), 0)
        gt = (blk3 > sim3).astype(jnp.float32)
        eq = jnp.where((blk3 == sim3) & (kvec < jmat), 1.0, 0.0)
        rank_t = rank_t + jnp.sum(gt + eq, axis=0)

    disc_t = jnp.log(rank_t + 2.0) * INV_LN2
    x = ap_t / disc_t

    m = jnp.max(x, axis=0, keepdims=True)
    z = x - m
    lse = jnp.log(jnp.sum(jnp.exp(z), axis=0, keepdims=True))
    logp = z - lse

    eye = (lax.broadcasted_iota(jnp.int32, (T_, T_), 0)
           == lax.broadcasted_iota(jnp.int32, (T_, T_), 1))
    present = counts > 0.0
    val = jnp.where(eye & present, -logp, 0.0)
    total = jnp.sum(jnp.sum(val, axis=1, keepdims=True), axis=0, keepdims=True)
    out_ref[...] = total / jnp.float32(T_)


@functools.partial(jax.jit, static_argnames=("interpret",))
def _run(features, labels, W, b, proto, temperature, interpret=False):
    x2 = features.reshape(N_, D_)
    lab2 = labels.reshape(LAB_ROWS, 1024).astype(jnp.int32)
    b2 = b.reshape(1, T_).astype(jnp.float32)
    temp = jnp.asarray(temperature, jnp.float32).reshape(1, 1)

    S = pl.pallas_call(
        _segsum_body,
        grid=(N_TILES,),
        in_specs=[
            pl.BlockSpec((LAB_ROWS, 1024), lambda g: (0, 0)),
            pl.BlockSpec((TOK_TILE, D_), lambda g: (g, 0)),
        ],
        out_specs=pl.BlockSpec((T_, D_), lambda g: (0, 0)),
        out_shape=jax.ShapeDtypeStruct((T_, D_), jnp.float32),
        interpret=interpret,
    )(lab2, x2)

    loss = pl.pallas_call(
        _epilogue_body,
        out_shape=jax.ShapeDtypeStruct((1, 1), jnp.float32),
        interpret=interpret,
    )(S, lab2, W.astype(jnp.float32), b2, proto.astype(jnp.float32), temp)
    return loss.reshape(1)


def kernel(features, labels, W, b, proto, temperature=0.3):
    return _run(features, labels, W, b, proto, temperature)
